# baseline (speedup 1.0000x reference)
import jax
import jax.numpy as jnp
from jax.experimental import pallas as pl

N = 10000
K = 32
D = 128
BLOCK = 400
SPLIT = 2
ROWS = BLOCK * K // SPLIT
SUB = BLOCK // SPLIT


def _body(x_ref, *rest):
    n_refs = rest[:SPLIT]
    wl_ref, wr_ref, b_ref, o_ref = rest[SPLIT:]
    means = [
        jnp.mean(r[...].reshape(SUB, K, D), axis=1) for r in n_refs
    ]
    mean = jnp.concatenate(means, axis=0)
    acc = jnp.dot(x_ref[...], wl_ref[...], preferred_element_type=jnp.float32)
    acc = acc + jnp.dot(mean, wr_ref[...], preferred_element_type=jnp.float32)
    o_ref[...] = acc + b_ref[...]


def kernel(x, neigh_x, W_l, b_l, W_r, b_r):
    wl_t = W_l.T
    wr_t = W_r.T
    bias = (b_l + b_r).reshape(1, D)
    nx2d = neigh_x.reshape(N * K, D)
    grid = (N // BLOCK,)

    def make_spec(s):
        return pl.BlockSpec((ROWS, D), lambda i, s=s: (i * SPLIT + s, 0))

    return pl.pallas_call(
        _body,
        grid=grid,
        in_specs=[
            pl.BlockSpec((BLOCK, D), lambda i: (i, 0)),
            *[make_spec(s) for s in range(SPLIT)],
            pl.BlockSpec((D, D), lambda i: (0, 0)),
            pl.BlockSpec((D, D), lambda i: (0, 0)),
            pl.BlockSpec((1, D), lambda i: (0, 0)),
        ],
        out_specs=pl.BlockSpec((BLOCK, D), lambda i: (i, 0)),
        out_shape=jax.ShapeDtypeStruct((N, D), jnp.float32),
    )(x, *([nx2d] * SPLIT), wl_t, wr_t, bias)


# speedup vs baseline: 1.0131x; 1.0131x over previous
import jax
import jax.numpy as jnp
from jax.experimental import pallas as pl

N = 10000
K = 32
D = 128
BLOCK = 400
SPLIT = 1
ROWS = BLOCK * K // SPLIT
SUB = BLOCK // SPLIT


def _body(x_ref, *rest):
    n_refs = rest[:SPLIT]
    wl_ref, wr_ref, b_ref, o_ref = rest[SPLIT:]
    means = [
        jnp.mean(r[...].reshape(SUB, K, D), axis=1) for r in n_refs
    ]
    mean = jnp.concatenate(means, axis=0)
    acc = jnp.dot(x_ref[...], wl_ref[...], preferred_element_type=jnp.float32)
    acc = acc + jnp.dot(mean, wr_ref[...], preferred_element_type=jnp.float32)
    o_ref[...] = acc + b_ref[...]


def kernel(x, neigh_x, W_l, b_l, W_r, b_r):
    wl_t = W_l.T
    wr_t = W_r.T
    bias = (b_l + b_r).reshape(1, D)
    nx2d = neigh_x.reshape(N * K, D)
    grid = (N // BLOCK,)

    def make_spec(s):
        return pl.BlockSpec((ROWS, D), lambda i, s=s: (i * SPLIT + s, 0))

    return pl.pallas_call(
        _body,
        grid=grid,
        in_specs=[
            pl.BlockSpec((BLOCK, D), lambda i: (i, 0)),
            *[make_spec(s) for s in range(SPLIT)],
            pl.BlockSpec((D, D), lambda i: (0, 0)),
            pl.BlockSpec((D, D), lambda i: (0, 0)),
            pl.BlockSpec((1, D), lambda i: (0, 0)),
        ],
        out_specs=pl.BlockSpec((BLOCK, D), lambda i: (i, 0)),
        out_shape=jax.ShapeDtypeStruct((N, D), jnp.float32),
    )(x, *([nx2d] * SPLIT), wl_t, wr_t, bias)
